# baseline (speedup 1.0000x reference)
import functools

import jax
import jax.numpy as jnp
from jax import lax
from jax.experimental import pallas as pl
from jax.experimental.pallas import tpu as pltpu
from jax.experimental.pallas import tpu_sc as plsc

N = 10000
D = 128
S = 32
NW = 32
CHUNK = 320
LANES = 16
NCH = D // LANES
NG = 4
DW = D // 2
BATCH = 4
BS = BATCH * S
NSTEPS = CHUNK // BATCH
NBUF = 4
QUARTER = CHUNK // 4
MM_BLK = 1000


def _mm_pack_body(x_ref, w_ref, o_ref):
    y = jnp.dot(x_ref[...], w_ref[...], preferred_element_type=jnp.float32)

    def rne16(i):
        return (i + 0x7FFF + ((i >> 16) & 1)) >> 16

    ilo = lax.bitcast_convert_type(y[:, :DW], jnp.int32)
    ihi = lax.bitcast_convert_type(y[:, DW:], jnp.int32)
    o_ref[...] = (rne16(ilo) & 0xFFFF) | (rne16(ihi) << 16)


def _matmul_pack(x, W):
    return pl.pallas_call(
        _mm_pack_body,
        grid=(N // MM_BLK,),
        in_specs=[pl.BlockSpec((MM_BLK, D), lambda i: (i, 0)),
                  pl.BlockSpec((D, D), lambda i: (0, 0))],
        out_specs=pl.BlockSpec((MM_BLK, DW), lambda i: (i, 0)),
        out_shape=jax.ShapeDtypeStruct((N, DW), jnp.int32),
    )(x, W)


def _unpack2(v):
    lo = lax.bitcast_convert_type(v << 16, jnp.float32)
    hi = lax.bitcast_convert_type(v, jnp.float32)
    return lo, hi


def _sc_gather_combine(ybi, adj_flat, b):
    mesh = plsc.VectorSubcoreMesh(core_axis_name="c", subcore_axis_name="s")

    @functools.partial(
        pl.kernel,
        mesh=mesh,
        compiler_params=pltpu.CompilerParams(needs_layout_passes=False,
                                             use_tc_tiling_on_sc=False),
        out_type=jax.ShapeDtypeStruct((N, D), jnp.float32),
        scratch_types=[
            pltpu.VMEM_SHARED((N, DW), jnp.int32),
            pltpu.VMEM((NSTEPS, BS), jnp.int32),
            pltpu.VMEM((CHUNK, DW), jnp.int32),
            pltpu.VMEM((BATCH, D), jnp.float32),
            pltpu.VMEM((BATCH, D), jnp.float32),
            pltpu.VMEM((BS, DW), jnp.int32),
            pltpu.VMEM((BS, DW), jnp.int32),
            pltpu.VMEM((BS, DW), jnp.int32),
            pltpu.VMEM((BS, DW), jnp.int32),
            pltpu.VMEM((D,), jnp.float32),
            pltpu.SemaphoreType.DMA,
            pltpu.SemaphoreType.DMA,
            pltpu.SemaphoreType.DMA,
            pltpu.SemaphoreType.DMA,
            pltpu.SemaphoreType.DMA,
            pltpu.SemaphoreType.DMA,
        ],
    )
    def k(yb_hbm, adj_hbm, b_hbm, out_hbm,
          yb_sp, adj_v, my_v, ob0, ob1, r0, r1, r2, r3, b_v,
          s0, s1, s2, s3, os0, os1):
        bufs = (r0, r1, r2, r3)
        sems = (s0, s1, s2, s3)
        obufs = (ob0, ob1)
        osems = (os0, os1)
        sid = lax.axis_index("s")
        wid = sid * 2 + lax.axis_index("c")
        base = wid * CHUNK
        sl16 = pl.ds(sid * (N // 16), N // 16)
        pltpu.sync_copy(yb_hbm.at[sl16], yb_sp.at[sl16])
        qsteps = QUARTER // BATCH
        for q in range(4):
            @pl.when(base + (q + 1) * QUARTER <= N)
            def _(q=q):
                src = pl.ds(base // BATCH + q * qsteps, qsteps)
                pltpu.sync_copy(adj_hbm.at[src],
                                adj_v.at[pl.ds(q * qsteps, qsteps)])

        pltpu.sync_copy(b_hbm, b_v)
        plsc.subcore_barrier()
        for q in range(4):
            @pl.when(base + (q + 1) * QUARTER <= N)
            def _(q=q):
                src = pl.ds(base + q * QUARTER, QUARTER)
                pltpu.sync_copy(yb_sp.at[src],
                                my_v.at[pl.ds(q * QUARTER, QUARTER)])

        def gather(s, buf, sem):
            idx = adj_v.at[s]
            return pltpu.make_async_copy(yb_sp.at[idx], buf, sem)

        def ocopy(s, q):
            dst = out_hbm.at[pl.ds(base + s * BATCH, BATCH)]
            return pltpu.make_async_copy(obufs[q], dst, osems[q])

        def accum(s, buf, ob):
            for t in range(BATCH):
                i = s * BATCH + t
                row = t * S

                def rowloop(jj, accs, row=row, buf=buf):
                    accs = list(accs)
                    r = row + jj * 8
                    for u in range(8):
                        for g in range(NG):
                            v = buf[r + u, pl.ds(g * LANES, LANES)]
                            lo, hi = _unpack2(v)
                            accs[g] = accs[g] + lo
                            accs[g + NG] = accs[g + NG] + hi
                    return tuple(accs)

                zero = jnp.zeros((LANES,), jnp.float32)
                accs = lax.fori_loop(0, S // 8, rowloop, (zero,) * NCH)
                for g in range(NG):
                    v = my_v[i, pl.ds(g * LANES, LANES)]
                    lo, hi = _unpack2(v)
                    sl0 = pl.ds(g * LANES, LANES)
                    sl1 = pl.ds((g + NG) * LANES, LANES)
                    ob[t, sl0] = 0.5 * lo + (0.5 / S) * accs[g] + b_v[sl0]
                    ob[t, sl1] = 0.5 * hi + (0.5 / S) * accs[g + NG] + b_v[sl1]

        trips = jnp.where(base + CHUNK <= N, NSTEPS // NBUF,
                          NSTEPS // NBUF // 4)
        for p in range(NBUF):
            gather(p, bufs[p], sems[p]).start()

        def body(g4, carry):
            for p in range(NBUF):
                s = NBUF * g4 + p
                q = p % 2
                gather(s, bufs[p], sems[p]).wait()

                @pl.when(s >= 2)
                def _(s=s, q=q):
                    ocopy(s - 2, q).wait()

                accum(s, bufs[p], obufs[q])
                ocopy(s, q).start()

                @pl.when((s < NSTEPS - NBUF) & (g4 < trips - 1))
                def _(s=s, p=p):
                    gather(s + NBUF, bufs[p], sems[p]).start()

            return carry

        lax.fori_loop(0, trips, body, 0)
        laststep = trips * NBUF
        ocopy(laststep - 2, 0).wait()
        ocopy(laststep - 1, 1).wait()

    return k(ybi, adj_flat, b)


def kernel(x, neighbor_adj, W, b):
    ybi = _matmul_pack(x, W)
    adj4 = neighbor_adj.astype(jnp.int32).reshape(N // BATCH, BS)
    return _sc_gather_combine(ybi, adj4, b)


# speedup vs baseline: 1.2658x; 1.2658x over previous
import functools

import jax
import jax.numpy as jnp
from jax import lax
from jax.experimental import pallas as pl
from jax.experimental.pallas import tpu as pltpu
from jax.experimental.pallas import tpu_sc as plsc

N = 10000
D = 128
S = 32
NW = 32
CHUNK = 320
LANES = 16
NCH = D // LANES
NG = 4
DW = D // 2
BATCH = 4
BS = BATCH * S
NSTEPS = CHUNK // BATCH
NBUF = 4
QUARTER = CHUNK // 4
MM_BLK = 1000


def _mm_pack_body(x_ref, w_ref, o_ref):
    y = jnp.dot(x_ref[...], w_ref[...], preferred_element_type=jnp.float32)

    def rne16(i):
        return (i + 0x7FFF + ((i >> 16) & 1)) >> 16

    ilo = lax.bitcast_convert_type(y[:, :DW], jnp.int32)
    ihi = lax.bitcast_convert_type(y[:, DW:], jnp.int32)
    o_ref[...] = (rne16(ilo) & 0xFFFF) | (rne16(ihi) << 16)


def _matmul_pack(x, W):
    return pl.pallas_call(
        _mm_pack_body,
        grid=(N // MM_BLK,),
        in_specs=[pl.BlockSpec((MM_BLK, D), lambda i: (i, 0)),
                  pl.BlockSpec((D, D), lambda i: (0, 0))],
        out_specs=pl.BlockSpec((MM_BLK, DW), lambda i: (i, 0)),
        out_shape=jax.ShapeDtypeStruct((N, DW), jnp.int32),
    )(x, W)


def _unpack2(v):
    lo = lax.bitcast_convert_type(v << 16, jnp.float32)
    hi = lax.bitcast_convert_type(v, jnp.float32)
    return lo, hi


def _sc_gather_combine(ybi, adj_flat, b):
    mesh = plsc.VectorSubcoreMesh(core_axis_name="c", subcore_axis_name="s")

    @functools.partial(
        pl.kernel,
        mesh=mesh,
        compiler_params=pltpu.CompilerParams(needs_layout_passes=False,
                                             use_tc_tiling_on_sc=False),
        out_type=jax.ShapeDtypeStruct((N, D), jnp.float32),
        scratch_types=[
            pltpu.VMEM_SHARED((N, DW), jnp.int32),
            pltpu.VMEM((NSTEPS, BS), jnp.int32),
            pltpu.VMEM((CHUNK, DW), jnp.int32),
            pltpu.VMEM((BATCH, D), jnp.float32),
            pltpu.VMEM((BATCH, D), jnp.float32),
            pltpu.VMEM((BS, DW), jnp.int32),
            pltpu.VMEM((BS, DW), jnp.int32),
            pltpu.VMEM((BS, DW), jnp.int32),
            pltpu.VMEM((BS, DW), jnp.int32),
            pltpu.VMEM((D,), jnp.float32),
            pltpu.SemaphoreType.DMA,
            pltpu.SemaphoreType.DMA,
            pltpu.SemaphoreType.DMA,
            pltpu.SemaphoreType.DMA,
            pltpu.SemaphoreType.DMA,
            pltpu.SemaphoreType.DMA,
        ],
    )
    def k(yb_hbm, adj_hbm, b_hbm, out_hbm,
          yb_sp, adj_v, my_v, ob0, ob1, r0, r1, r2, r3, b_v,
          s0, s1, s2, s3, os0, os1):
        bufs = (r0, r1, r2, r3)
        sems = (s0, s1, s2, s3)
        obufs = (ob0, ob1)
        osems = (os0, os1)
        sid = lax.axis_index("s")
        wid = sid * 2 + lax.axis_index("c")
        base = wid * CHUNK
        sl16 = pl.ds(sid * (N // 16), N // 16)
        pltpu.sync_copy(yb_hbm.at[sl16], yb_sp.at[sl16])
        qsteps = QUARTER // BATCH
        for q in range(4):
            @pl.when(base + (q + 1) * QUARTER <= N)
            def _(q=q):
                src = pl.ds(base // BATCH + q * qsteps, qsteps)
                pltpu.sync_copy(adj_hbm.at[src],
                                adj_v.at[pl.ds(q * qsteps, qsteps)])

        pltpu.sync_copy(b_hbm, b_v)
        plsc.subcore_barrier()
        for q in range(4):
            @pl.when(base + (q + 1) * QUARTER <= N)
            def _(q=q):
                src = pl.ds(base + q * QUARTER, QUARTER)
                pltpu.sync_copy(yb_sp.at[src],
                                my_v.at[pl.ds(q * QUARTER, QUARTER)])

        def gather(s, buf, sem):
            idx = adj_v.at[s]
            return pltpu.make_async_copy(yb_sp.at[idx], buf, sem)

        def ocopy(s, q):
            dst = out_hbm.at[pl.ds(base + s * BATCH, BATCH)]
            return pltpu.make_async_copy(obufs[q], dst, osems[q])

        def accum(s, buf, ob):
            for t in range(BATCH):
                i = s * BATCH + t
                row = t * S

                def rowloop(jj, accs, row=row, buf=buf):
                    accs = list(accs)
                    r = row + jj * 4
                    for u in range(4):
                        for g in range(NG):
                            v = buf[r + u, pl.ds(g * LANES, LANES)]
                            lo, hi = _unpack2(v)
                            accs[g] = accs[g] + lo
                            accs[g + NG] = accs[g + NG] + hi
                    return tuple(accs)

                zero = jnp.zeros((LANES,), jnp.float32)
                accs = lax.fori_loop(0, S // 4, rowloop, (zero,) * NCH)
                for g in range(NG):
                    v = my_v[i, pl.ds(g * LANES, LANES)]
                    lo, hi = _unpack2(v)
                    sl0 = pl.ds(g * LANES, LANES)
                    sl1 = pl.ds((g + NG) * LANES, LANES)
                    ob[t, sl0] = 0.5 * lo + (0.5 / S) * accs[g] + b_v[sl0]
                    ob[t, sl1] = 0.5 * hi + (0.5 / S) * accs[g + NG] + b_v[sl1]

        trips = jnp.where(base + CHUNK <= N, NSTEPS // NBUF,
                          NSTEPS // NBUF // 4)
        for p in range(NBUF):
            gather(p, bufs[p], sems[p]).start()

        def body(g4, carry):
            for p in range(NBUF):
                s = NBUF * g4 + p
                q = p % 2
                gather(s, bufs[p], sems[p]).wait()

                @pl.when(s >= 2)
                def _(s=s, q=q):
                    ocopy(s - 2, q).wait()

                accum(s, bufs[p], obufs[q])
                ocopy(s, q).start()

                @pl.when((s < NSTEPS - NBUF) & (g4 < trips - 1))
                def _(s=s, p=p):
                    gather(s + NBUF, bufs[p], sems[p]).start()

            return carry

        lax.fori_loop(0, trips, body, 0)
        laststep = trips * NBUF
        ocopy(laststep - 2, 0).wait()
        ocopy(laststep - 1, 1).wait()

    return k(ybi, adj_flat, b)


def kernel(x, neighbor_adj, W, b):
    ybi = _matmul_pack(x, W)
    adj4 = neighbor_adj.astype(jnp.int32).reshape(N // BATCH, BS)
    return _sc_gather_combine(ybi, adj4, b)
